# Initial kernel scaffold; baseline (speedup 1.0000x reference)
import jax
import jax.numpy as jnp
from jax.experimental import pallas as pl


def kernel(xr, xp, sr, sp, params, rp_edge_index, rr_edge_index, pp_edge_index, samples, labels):
    raise NotImplementedError("write your pallas kernel here")


# speedup vs baseline: 1.0025x; 1.0025x over previous
import functools

import jax
import jax.numpy as jnp
from jax.experimental import pallas as pl
from jax.experimental.pallas import tpu as pltpu

NR = 25000
NP_ = 25000
N = NR + NP_
D = 128
H = 16
B = 16384


def _gcn_conv(x, edge_index, edge_weight, W, b, num_nodes):
    src = edge_index[0]
    dst = edge_index[1]
    E = src.shape[0]
    if edge_weight is None:
        edge_weight = jnp.ones((E,), x.dtype)
    loop = jnp.arange(num_nodes)
    src = jnp.concatenate([src, loop])
    dst = jnp.concatenate([dst, loop])
    ew = jnp.concatenate([edge_weight, jnp.ones((num_nodes,), x.dtype)])
    deg = jax.ops.segment_sum(ew, dst, num_segments=num_nodes)
    dis = jnp.where(deg > 0, 1.0 / jnp.sqrt(deg), 0.0)
    norm = dis[src] * ew * dis[dst]
    xw = x @ W.T
    msg = xw[src] * norm[:, None]
    out = jax.ops.segment_sum(msg, dst, num_segments=num_nodes) + b
    return out


def _encoder_fwd(x, ei, ew, layers, n):
    x1 = jax.nn.relu(_gcn_conv(x, ei, ew, layers[0]["W"], layers[0]["b"], n))
    x2 = jax.nn.relu(_gcn_conv(x1, ei, ew, layers[1]["W"], layers[1]["b"], n))
    return jax.nn.relu(_gcn_conv(x2, ei, ew, layers[2]["W"], layers[2]["b"], n))


def _attention_fwd(z, p):
    w = jnp.tanh(z @ p["W1"].T + p["b1"]) @ p["W2"].T
    beta = jax.nn.softmax(w, axis=1)
    return (beta * z).sum(1)


def _mlp_head_kernel(u_ref, v_ref, w1u_ref, w1v_ref, b1_ref, g_ref, be_ref,
                     w2_ref, b2_ref, o_ref):
    h = (u_ref[...] @ w1u_ref[...] + v_ref[...] @ w1v_ref[...]) + b1_ref[...]
    h = h * g_ref[...] + be_ref[...]
    h = jnp.maximum(h, 0.0)
    o = jnp.dot(h, w2_ref[...])
    o_ref[...] = jax.nn.sigmoid(o + b2_ref[0, 0])


def _mlp_head(u, v, p):
    bn = p["gamma"] / jnp.sqrt(1.0 + 1e-5)
    w1 = p["W1"].T
    w1u = w1[:D]
    w1v = w1[D:]
    out = pl.pallas_call(
        _mlp_head_kernel,
        grid=(B // 2048,),
        in_specs=[
            pl.BlockSpec((2048, D), lambda i: (i, 0)),
            pl.BlockSpec((2048, D), lambda i: (i, 0)),
            pl.BlockSpec((D, D), lambda i: (0, 0)),
            pl.BlockSpec((D, D), lambda i: (0, 0)),
            pl.BlockSpec((1, D), lambda i: (0, 0)),
            pl.BlockSpec((1, D), lambda i: (0, 0)),
            pl.BlockSpec((1, D), lambda i: (0, 0)),
            pl.BlockSpec((D, 1), lambda i: (0, 0)),
            pl.BlockSpec((1, 1), lambda i: (0, 0)),
        ],
        out_specs=pl.BlockSpec((2048, 1), lambda i: (i, 0)),
        out_shape=jax.ShapeDtypeStruct((B, 1), jnp.float32),
    )(u, v, w1u, w1v, p["b1"][None, :], bn[None, :], p["beta"][None, :],
      p["W2"].T, p["b2"][None, :])
    return out[:, 0]


def _bce(p, y):
    p = jnp.clip(p, 1e-7, 1.0 - 1e-7)
    return -jnp.mean(y * jnp.log(p) + (1.0 - y) * jnp.log(1.0 - p))


def kernel(xr, xp, sr, sp, params, rp_edge_index, rr_edge_index, pp_edge_index, samples, labels):
    rr_xr = xr @ params["rr_xr_proj"]["W"].T + params["rr_xr_proj"]["b"]
    pp_xp = xp @ params["pp_xp_proj"]["W"].T + params["pp_xp_proj"]["b"]
    rp_xr = xr @ params["rp_xr_proj"]["W"].T + params["rp_xr_proj"]["b"]
    rp_xp = xp @ params["rp_xp_proj"]["W"].T + params["rp_xp_proj"]["b"]
    rp_x = jnp.concatenate([rp_xr, rp_xp])
    rr_hr = _encoder_fwd(rr_xr, rr_edge_index, sr, params["rr_enc"], NR)
    rp_h = _encoder_fwd(rp_x, rp_edge_index, None, params["rp_enc"], N)
    pp_hp = _encoder_fwd(pp_xp, pp_edge_index, sp, params["pp_enc"], NP_)
    rp_hr = rp_h[:NR]
    rp_hp = rp_h[NR:]
    hr = _attention_fwd(jnp.stack([rr_hr, rp_hr], axis=1), params["r_at"])
    hp = _attention_fwd(jnp.stack([pp_hp, rp_hp], axis=1), params["p_at"])
    z = jnp.concatenate([hr, hp])
    u = z[samples[:, 0]]
    v = z[samples[:, 1]]
    out = _mlp_head(u, v, params["mlp"])
    pred_loss = _bce(out, labels)
    u_i = rp_h[samples[:, 0]]
    v_i = rp_h[samples[:, 1]]
    out_i = _mlp_head(u_i, v_i, params["i_mlp"])
    pred_i_loss = _bce(out_i, labels)
    z_c = jnp.concatenate([rr_hr, pp_hp])
    u_c = z_c[samples[:, 0]]
    v_c = z_c[samples[:, 1]]
    out_c = _mlp_head(u_c, v_c, params["c_mlp"])
    pred_c_loss = _bce(out_c, labels)
    loss = pred_loss + 0.5 * (pred_i_loss + pred_c_loss)
    return (out, loss, rr_hr, rp_hr, pp_hp, rp_hp)


# speedup vs baseline: 3.3239x; 3.3154x over previous
import functools

import jax
import jax.numpy as jnp
from jax import lax
from jax.experimental import pallas as pl
from jax.experimental.pallas import tpu as pltpu
from jax.experimental.pallas import tpu_sc as plsc

NR = 25000
NP_ = 25000
N = NR + NP_
D = 128
B = 16384

NC = 2
NS = 16
NW = NC * NS
KK = 2

_RR_CFG = dict(n=25000, npad=25088, Dc=32, P=4, nch=102)
_RP_CFG = dict(n=50000, npad=50176, Dc=16, P=8, nch=156)

_DEG_REG = (25088, 25088, 50176)
_DEG_OFF = (0, 25088, 50176)
_DEG_TOT = 100352
_DEG_NCH = 360


def _mesh():
    return plsc.VectorSubcoreMesh(
        core_axis_name="c", subcore_axis_name="s",
        num_cores=NC, num_subcores=NS)


def _zero_vmem_1d(ref, nwords):
    z = jnp.zeros((16,), jnp.float32)

    def body(i, _):
        ref[pl.ds(i * 16, 16)] = z
        return 0

    lax.fori_loop(0, nwords // 16, body, 0)


def _zero_vmem_2d(ref, rows, cols):
    z = jnp.zeros((16,), jnp.float32)

    def body(i, _):
        for k in range(cols // 16):
            ref[i, pl.ds(k * 16, 16)] = z
        return 0

    lax.fori_loop(0, rows, body, 0)


def _sc_degrees(dst_cat, w_cat):
    stride = _DEG_TOT // NS

    def body(dst_hbm, w_hbm, out_hbm, dst_v, w_v, zbuf, acc):
        c = lax.axis_index("c")
        s = lax.axis_index("s")
        wid = c * NS + s
        pltpu.sync_copy(dst_hbm.at[wid], dst_v)
        pltpu.sync_copy(w_hbm.at[wid], w_v)
        _zero_vmem_1d(zbuf, stride)
        pltpu.sync_copy(zbuf, acc.at[pl.ds(s * stride, stride)])
        plsc.subcore_barrier()

        def chunk(j, _):
            pltpu.sync_copy(w_v.at[j], acc.at[dst_v.at[j]], add=True)
            return 0

        lax.fori_loop(0, _DEG_NCH, chunk, 0)
        plsc.subcore_barrier()
        pltpu.sync_copy(acc.at[pl.ds(s * stride, stride)],
                        out_hbm.at[c, pl.ds(s * stride, stride)])

    f = pl.kernel(
        body,
        out_type=jax.ShapeDtypeStruct((NC, _DEG_TOT), jnp.float32),
        mesh=_mesh(),
        scratch_types=[
            pltpu.VMEM((_DEG_NCH, 128), jnp.int32),
            pltpu.VMEM((_DEG_NCH, 128), jnp.float32),
            pltpu.VMEM((stride,), jnp.float32),
            pltpu.VMEM_SHARED((_DEG_TOT,), jnp.float32),
        ],
    )
    return f(dst_cat, w_cat)


def _make_scatter(n, npad, Dc, P, nch, weighted):
    stride = npad // NS
    zrows = 392
    nz = stride // zrows
    nsup = nch // KK
    T = nsup // 3
    nrow_idx = nch + KK

    def fire_g(xs2_hbm, srcp_v, rows, sem, s):
        cps = []
        for k in range(KK):
            cps.append(pltpu.async_copy(
                xs2_hbm.at[srcp_v.at[s * KK + k]],
                rows.at[pl.ds(k * 128, 128)], sem))
        return cps

    def wait_g(xs2_hbm, srcp_v, rows, sem):
        for k in range(KK):
            pltpu.make_async_copy(
                xs2_hbm.at[srcp_v.at[k]],
                rows.at[pl.ds(k * 128, 128)], sem).wait()

    def fire_s(acc, dst_v, rows, sem, s):
        for k in range(KK):
            pltpu.async_copy(
                rows.at[pl.ds(k * 128, 128)],
                acc.at[dst_v.at[s * KK + k]], sem, add=True)

    def wait_s(acc, dst_v, rows, sem):
        for k in range(KK):
            pltpu.make_async_copy(
                rows.at[pl.ds(k * 128, 128)],
                acc.at[dst_v.at[k]], sem).wait()

    def mult(rows, w_v, s):
        for k in range(KK):
            j = s * KK + k

            def body(g, _):
                wv16 = w_v[j, pl.ds(g * 16, 16)]
                base = k * 128 + g * 16
                for u in range(16):
                    wsp = wv16.at[jnp.full((16,), u, jnp.int32)].get(
                        mode="promise_in_bounds")
                    for q in range(Dc // 16):
                        sl = pl.ds(q * 16, 16)
                        rows[base + u, sl] = rows[base + u, sl] * wsp
                return 0

            lax.fori_loop(0, 8, body, 0)

    def body(xs2_hbm, srcp_hbm, dst_hbm, *rest):
        if weighted:
            w_hbm = rest[0]
            rest = rest[1:]
        (out_hbm, srcp_v, dst_v) = rest[:3]
        rest = rest[3:]
        if weighted:
            w_v = rest[0]
            rest = rest[1:]
        (r0, r1, r2, zbuf, acc, g0, g1, g2, s0, s1, s2) = rest
        rows = (r0, r1, r2)
        gsem = (g0, g1, g2)
        ssem = (s0, s1, s2)
        c = lax.axis_index("c")
        s = lax.axis_index("s")
        wid = c * NS + s
        pltpu.sync_copy(srcp_hbm.at[wid], srcp_v)
        pltpu.sync_copy(dst_hbm.at[wid], dst_v)
        if weighted:
            pltpu.sync_copy(w_hbm.at[wid], w_v)
        _zero_vmem_2d(zbuf, zrows, Dc)

        def phase(sidx, X, do_wait_s):
            Y = (X + 1) % 3
            wait_g(xs2_hbm, srcp_v, rows[X], gsem[X])
            if weighted:
                mult(rows[X], w_v, sidx)
            fire_s(acc, dst_v, rows[X], ssem[X], sidx)
            if do_wait_s:
                wait_s(acc, dst_v, rows[Y], ssem[Y])
            fire_g(xs2_hbm, srcp_v, rows[Y], gsem[Y], sidx + 1)

        def incr_srcp():
            one = jnp.full((16,), 1, jnp.int32)

            def incr(i, _):
                for q in range(8):
                    sl = pl.ds(q * 16, 16)
                    srcp_v[i, sl] = srcp_v[i, sl] + one
                return 0

            lax.fori_loop(0, nrow_idx, incr, 0)

        def do_pass(p):
            fire_g(xs2_hbm, srcp_v, rows[0], gsem[0], 0)
            for z in range(nz):
                pltpu.sync_copy(
                    zbuf, acc.at[pl.ds(s * stride + z * zrows, zrows)])
            plsc.subcore_barrier()
            phase(0, 0, False)
            phase(1, 1, False)
            phase(2, 2, True)

            def ring(t, _):
                sb = t * 3
                phase(sb, 0, True)
                phase(sb + 1, 1, True)
                phase(sb + 2, 2, True)
                return 0

            lax.fori_loop(1, T, ring, 0)
            wait_s(acc, dst_v, rows[1], ssem[1])
            wait_s(acc, dst_v, rows[2], ssem[2])
            wait_g(xs2_hbm, srcp_v, rows[0], gsem[0])
            plsc.subcore_barrier()
            pltpu.sync_copy(
                acc.at[pl.ds(s * stride, stride)],
                out_hbm.at[c, p, pl.ds(s * stride, stride)])
            plsc.subcore_barrier()

        do_pass(0)

        def later(p, _):
            incr_srcp()
            do_pass(p)
            return 0

        lax.fori_loop(1, P, later, 0)

    scratch = [
        pltpu.VMEM((nrow_idx, 128), jnp.int32),
        pltpu.VMEM((nrow_idx, 128), jnp.int32),
    ]
    if weighted:
        scratch.append(pltpu.VMEM((nrow_idx, 128), jnp.float32))
    scratch += [
        pltpu.VMEM((KK * 128, Dc), jnp.float32),
        pltpu.VMEM((KK * 128, Dc), jnp.float32),
        pltpu.VMEM((KK * 128, Dc), jnp.float32),
        pltpu.VMEM((zrows, Dc), jnp.float32),
        pltpu.VMEM_SHARED((npad, Dc), jnp.float32),
    ] + [pltpu.SemaphoreType.DMA] * 6

    return pl.kernel(
        body,
        out_type=jax.ShapeDtypeStruct((NC, P, npad, Dc), jnp.float32),
        mesh=_mesh(),
        scratch_types=scratch,
        compiler_params=pltpu.CompilerParams(use_tc_tiling_on_sc=False),
    )


_SCATTER_CACHE = {}


def _scatter(xs2, srcp, dst, w, cfg):
    key = (cfg["n"], cfg["Dc"], cfg["P"], cfg["nch"], w is not None)
    if key not in _SCATTER_CACHE:
        _SCATTER_CACHE[key] = _make_scatter(
            cfg["n"], cfg["npad"], cfg["Dc"], cfg["P"], cfg["nch"],
            w is not None)
    f = _SCATTER_CACHE[key]
    if w is not None:
        return f(xs2, srcp, dst, w)
    return f(xs2, srcp, dst)


def _prep_edges(ei, ew, cfg):
    n, P, nch = cfg["n"], cfg["P"], cfg["nch"]
    E = ei.shape[1]
    e_pad = NW * nch * 128 - E
    srcp = jnp.concatenate(
        [ei[0] * P, jnp.zeros((e_pad,), jnp.int32)]).reshape(NW, nch, 128)
    dstp = jnp.concatenate(
        [ei[1], jnp.full((e_pad,), n, jnp.int32)]).reshape(NW, nch, 128)
    zrow = jnp.zeros((NW, KK, 128), jnp.int32)
    srcp = jnp.concatenate([srcp, zrow], axis=1)
    dstp = jnp.concatenate([dstp, jnp.full((NW, KK, 128), n, jnp.int32)],
                           axis=1)
    if ew is None:
        return srcp, dstp, None
    wp = jnp.concatenate(
        [ew, jnp.zeros((e_pad,), jnp.float32)]).reshape(NW, nch, 128)
    wp = jnp.concatenate([wp, jnp.zeros((NW, KK, 128), jnp.float32)], axis=1)
    return srcp, dstp, wp


def _gcn_layer(x, W, b, dis, srcp, dstp, wp, cfg):
    n, npad, Dc, P = cfg["n"], cfg["npad"], cfg["Dc"], cfg["P"]
    xw = x @ W.T
    xs2 = (xw * dis[:, None]).reshape(n * P, Dc)
    parts = _scatter(xs2, srcp, dstp, wp, cfg)
    agg = (parts[0] + parts[1])[:, :n, :]
    agg = agg.transpose(1, 0, 2).reshape(n, D)
    out = dis[:, None] * agg + (dis * dis)[:, None] * xw + b
    return jax.nn.relu(out)


def _encoder(x, layers, dis, srcp, dstp, wp, cfg):
    for i in range(3):
        x = _gcn_layer(x, layers[i]["W"], layers[i]["b"], dis,
                       srcp, dstp, wp, cfg)
    return x


def _attention_fwd(z, p):
    w = jnp.tanh(z @ p["W1"].T + p["b1"]) @ p["W2"].T
    beta = jax.nn.softmax(w, axis=1)
    return (beta * z).sum(1)


def _mlp_head_kernel(u_ref, v_ref, w1u_ref, w1v_ref, b1_ref, g_ref, be_ref,
                     w2_ref, b2_ref, o_ref):
    h = (u_ref[...] @ w1u_ref[...] + v_ref[...] @ w1v_ref[...]) + b1_ref[...]
    h = h * g_ref[...] + be_ref[...]
    h = jnp.maximum(h, 0.0)
    o = jnp.dot(h, w2_ref[...])
    o_ref[...] = jax.nn.sigmoid(o + b2_ref[0, 0])


def _mlp_head(u, v, p):
    bn = p["gamma"] / jnp.sqrt(1.0 + 1e-5)
    w1 = p["W1"].T
    out = pl.pallas_call(
        _mlp_head_kernel,
        grid=(B // 2048,),
        in_specs=[
            pl.BlockSpec((2048, D), lambda i: (i, 0)),
            pl.BlockSpec((2048, D), lambda i: (i, 0)),
            pl.BlockSpec((D, D), lambda i: (0, 0)),
            pl.BlockSpec((D, D), lambda i: (0, 0)),
            pl.BlockSpec((1, D), lambda i: (0, 0)),
            pl.BlockSpec((1, D), lambda i: (0, 0)),
            pl.BlockSpec((1, D), lambda i: (0, 0)),
            pl.BlockSpec((D, 1), lambda i: (0, 0)),
            pl.BlockSpec((1, 1), lambda i: (0, 0)),
        ],
        out_specs=pl.BlockSpec((2048, 1), lambda i: (i, 0)),
        out_shape=jax.ShapeDtypeStruct((B, 1), jnp.float32),
    )(u, v, w1[:D], w1[D:], p["b1"][None, :], bn[None, :],
      p["beta"][None, :], p["W2"].T, p["b2"][None, :])
    return out[:, 0]


def _bce(p, y):
    p = jnp.clip(p, 1e-7, 1.0 - 1e-7)
    return -jnp.mean(y * jnp.log(p) + (1.0 - y) * jnp.log(1.0 - p))


def kernel(xr, xp, sr, sp, params, rp_edge_index, rr_edge_index,
           pp_edge_index, samples, labels):
    rr_s, rr_d, rr_w = _prep_edges(rr_edge_index, sr, _RR_CFG)
    pp_s, pp_d, pp_w = _prep_edges(pp_edge_index, sp, _RR_CFG)
    rp_s, rp_d, _ = _prep_edges(rp_edge_index, None, _RP_CFG)

    def _deg_slice(ei, ew, cfg, off):
        E = ei.shape[1]
        e_pad = NW * cfg["nch"] * 128 - E
        dd = jnp.concatenate(
            [ei[1], jnp.full((e_pad,), cfg["n"], jnp.int32)]) + off
        if ew is None:
            ww = jnp.concatenate(
                [jnp.ones((E,), jnp.float32), jnp.zeros((e_pad,))])
        else:
            ww = jnp.concatenate([ew, jnp.zeros((e_pad,), jnp.float32)])
        return dd, ww

    d0, w0 = _deg_slice(rr_edge_index, sr, _RR_CFG, _DEG_OFF[0])
    d1, w1 = _deg_slice(pp_edge_index, sp, _RR_CFG, _DEG_OFF[1])
    d2, w2 = _deg_slice(rp_edge_index, None, _RP_CFG, _DEG_OFF[2])
    dst_cat = jnp.concatenate([d0, d1, d2]).reshape(NW, _DEG_NCH, 128)
    w_cat = jnp.concatenate([w0, w1, w2]).reshape(NW, _DEG_NCH, 128)
    degp = _sc_degrees(dst_cat, w_cat)
    deg = degp[0] + degp[1]
    dis_rr = lax.rsqrt(deg[_DEG_OFF[0]:_DEG_OFF[0] + NR] + 1.0)
    dis_pp = lax.rsqrt(deg[_DEG_OFF[1]:_DEG_OFF[1] + NP_] + 1.0)
    dis_rp = lax.rsqrt(deg[_DEG_OFF[2]:_DEG_OFF[2] + N] + 1.0)

    rr_xr = xr @ params["rr_xr_proj"]["W"].T + params["rr_xr_proj"]["b"]
    pp_xp = xp @ params["pp_xp_proj"]["W"].T + params["pp_xp_proj"]["b"]
    rp_xr = xr @ params["rp_xr_proj"]["W"].T + params["rp_xr_proj"]["b"]
    rp_xp = xp @ params["rp_xp_proj"]["W"].T + params["rp_xp_proj"]["b"]
    rp_x = jnp.concatenate([rp_xr, rp_xp])

    rr_hr = _encoder(rr_xr, params["rr_enc"], dis_rr, rr_s, rr_d, rr_w,
                     _RR_CFG)
    rp_h = _encoder(rp_x, params["rp_enc"], dis_rp, rp_s, rp_d, None,
                    _RP_CFG)
    pp_hp = _encoder(pp_xp, params["pp_enc"], dis_pp, pp_s, pp_d, pp_w,
                     _RR_CFG)
    rp_hr = rp_h[:NR]
    rp_hp = rp_h[NR:]

    hr = _attention_fwd(jnp.stack([rr_hr, rp_hr], axis=1), params["r_at"])
    hp = _attention_fwd(jnp.stack([pp_hp, rp_hp], axis=1), params["p_at"])
    z = jnp.concatenate([hr, hp])
    u = z[samples[:, 0]]
    v = z[samples[:, 1]]
    out = _mlp_head(u, v, params["mlp"])
    pred_loss = _bce(out, labels)
    u_i = rp_h[samples[:, 0]]
    v_i = rp_h[samples[:, 1]]
    out_i = _mlp_head(u_i, v_i, params["i_mlp"])
    pred_i_loss = _bce(out_i, labels)
    z_c = jnp.concatenate([rr_hr, pp_hp])
    u_c = z_c[samples[:, 0]]
    v_c = z_c[samples[:, 1]]
    out_c = _mlp_head(u_c, v_c, params["c_mlp"])
    pred_c_loss = _bce(out_c, labels)
    loss = pred_loss + 0.5 * (pred_i_loss + pred_c_loss)
    return (out, loss, rr_hr, rp_hr, pp_hp, rp_hp)


# speedup vs baseline: 3.4135x; 1.0270x over previous
import functools

import jax
import jax.numpy as jnp
from jax import lax
from jax.experimental import pallas as pl
from jax.experimental.pallas import tpu as pltpu
from jax.experimental.pallas import tpu_sc as plsc

NR = 25000
NP_ = 25000
N = NR + NP_
D = 128
B = 16384

NC = 2
NS = 16
NW = NC * NS
KK = 2
H_AT = 16

_RR_CFG = dict(n=25000, npad=25088, Dc=32, P=4, nch=102)
_RP_CFG = dict(n=50000, npad=50176, Dc=16, P=8, nch=156)

_DEG_REG = (25088, 25088, 50176)
_DEG_OFF = (0, 25088, 50176)
_DEG_TOT = 100352
_DEG_NCH = 360


def _mesh():
    return plsc.VectorSubcoreMesh(
        core_axis_name="c", subcore_axis_name="s",
        num_cores=NC, num_subcores=NS)


def _zero_vmem_1d(ref, nwords):
    z = jnp.zeros((16,), jnp.float32)

    def body(i, _):
        ref[pl.ds(i * 16, 16)] = z
        return 0

    lax.fori_loop(0, nwords // 16, body, 0)


def _zero_vmem_2d(ref, rows, cols):
    z = jnp.zeros((16,), jnp.float32)

    def body(i, _):
        for k in range(cols // 16):
            ref[i, pl.ds(k * 16, 16)] = z
        return 0

    lax.fori_loop(0, rows, body, 0)


def _sc_degrees(dst_cat, w_cat):
    stride = _DEG_TOT // NS

    def body(dst_hbm, w_hbm, out_hbm, dst_v, w_v, zbuf, acc):
        c = lax.axis_index("c")
        s = lax.axis_index("s")
        wid = c * NS + s
        pltpu.sync_copy(dst_hbm.at[wid], dst_v)
        pltpu.sync_copy(w_hbm.at[wid], w_v)
        _zero_vmem_1d(zbuf, stride)
        pltpu.sync_copy(zbuf, acc.at[pl.ds(s * stride, stride)])
        plsc.subcore_barrier()

        def chunk(j, _):
            pltpu.sync_copy(w_v.at[j], acc.at[dst_v.at[j]], add=True)
            return 0

        lax.fori_loop(0, _DEG_NCH, chunk, 0)
        plsc.subcore_barrier()
        pltpu.sync_copy(acc.at[pl.ds(s * stride, stride)],
                        out_hbm.at[c, pl.ds(s * stride, stride)])

    f = pl.kernel(
        body,
        out_type=jax.ShapeDtypeStruct((NC, _DEG_TOT), jnp.float32),
        mesh=_mesh(),
        scratch_types=[
            pltpu.VMEM((_DEG_NCH, 128), jnp.int32),
            pltpu.VMEM((_DEG_NCH, 128), jnp.float32),
            pltpu.VMEM((stride,), jnp.float32),
            pltpu.VMEM_SHARED((_DEG_TOT,), jnp.float32),
        ],
    )
    return f(dst_cat, w_cat)


def _make_scatter(n, npad, Dc, P, nch, weighted):
    stride = npad // NS
    zrows = 392
    nz = stride // zrows
    nsup = nch // KK
    T = nsup // 3
    nrow_idx = nch + KK

    def fire_g(xs2_hbm, srcp_v, rows, sem, s):
        cps = []
        for k in range(KK):
            cps.append(pltpu.async_copy(
                xs2_hbm.at[srcp_v.at[s * KK + k]],
                rows.at[pl.ds(k * 128, 128)], sem))
        return cps

    def wait_g(xs2_hbm, srcp_v, rows, sem):
        for k in range(KK):
            pltpu.make_async_copy(
                xs2_hbm.at[srcp_v.at[k]],
                rows.at[pl.ds(k * 128, 128)], sem).wait()

    def fire_s(acc, dst_v, rows, sem, s):
        for k in range(KK):
            pltpu.async_copy(
                rows.at[pl.ds(k * 128, 128)],
                acc.at[dst_v.at[s * KK + k]], sem, add=True)

    def wait_s(acc, dst_v, rows, sem):
        for k in range(KK):
            pltpu.make_async_copy(
                rows.at[pl.ds(k * 128, 128)],
                acc.at[dst_v.at[k]], sem).wait()

    def mult(rows, w_v, s):
        for k in range(KK):
            j = s * KK + k

            def body(g, _):
                wv16 = w_v[j, pl.ds(g * 16, 16)]
                base = k * 128 + g * 16
                for u in range(16):
                    wsp = wv16.at[jnp.full((16,), u, jnp.int32)].get(
                        mode="promise_in_bounds")
                    for q in range(Dc // 16):
                        sl = pl.ds(q * 16, 16)
                        rows[base + u, sl] = rows[base + u, sl] * wsp
                return 0

            lax.fori_loop(0, 8, body, 0)

    def body(xs2_hbm, srcp_hbm, dst_hbm, *rest):
        if weighted:
            w_hbm = rest[0]
            rest = rest[1:]
        (out_hbm, srcp_v, dst_v) = rest[:3]
        rest = rest[3:]
        if weighted:
            w_v = rest[0]
            rest = rest[1:]
        (r0, r1, r2, zbuf, acc, g0, g1, g2, s0, s1, s2) = rest
        rows = (r0, r1, r2)
        gsem = (g0, g1, g2)
        ssem = (s0, s1, s2)
        c = lax.axis_index("c")
        s = lax.axis_index("s")
        wid = c * NS + s
        pltpu.sync_copy(srcp_hbm.at[wid], srcp_v)
        pltpu.sync_copy(dst_hbm.at[wid], dst_v)
        if weighted:
            pltpu.sync_copy(w_hbm.at[wid], w_v)
        _zero_vmem_2d(zbuf, zrows, Dc)

        def phase(sidx, X, do_wait_s):
            Y = (X + 1) % 3
            wait_g(xs2_hbm, srcp_v, rows[X], gsem[X])
            if weighted:
                mult(rows[X], w_v, sidx)
            fire_s(acc, dst_v, rows[X], ssem[X], sidx)
            if do_wait_s:
                wait_s(acc, dst_v, rows[Y], ssem[Y])
            fire_g(xs2_hbm, srcp_v, rows[Y], gsem[Y], sidx + 1)

        def incr_srcp():
            one = jnp.full((16,), 1, jnp.int32)

            def incr(i, _):
                for q in range(8):
                    sl = pl.ds(q * 16, 16)
                    srcp_v[i, sl] = srcp_v[i, sl] + one
                return 0

            lax.fori_loop(0, nrow_idx, incr, 0)

        def do_pass(p):
            fire_g(xs2_hbm, srcp_v, rows[0], gsem[0], 0)
            for z in range(nz):
                pltpu.sync_copy(
                    zbuf, acc.at[pl.ds(s * stride + z * zrows, zrows)])
            plsc.subcore_barrier()
            phase(0, 0, False)
            phase(1, 1, False)
            phase(2, 2, True)

            def ring(t, _):
                sb = t * 3
                phase(sb, 0, True)
                phase(sb + 1, 1, True)
                phase(sb + 2, 2, True)
                return 0

            lax.fori_loop(1, T, ring, 0)
            wait_s(acc, dst_v, rows[1], ssem[1])
            wait_s(acc, dst_v, rows[2], ssem[2])
            wait_g(xs2_hbm, srcp_v, rows[0], gsem[0])
            plsc.subcore_barrier()
            pltpu.sync_copy(
                acc.at[pl.ds(s * stride, stride)],
                out_hbm.at[c, p, pl.ds(s * stride, stride)])
            plsc.subcore_barrier()

        do_pass(0)

        def later(p, _):
            incr_srcp()
            do_pass(p)
            return 0

        lax.fori_loop(1, P, later, 0)

    scratch = [
        pltpu.VMEM((nrow_idx, 128), jnp.int32),
        pltpu.VMEM((nrow_idx, 128), jnp.int32),
    ]
    if weighted:
        scratch.append(pltpu.VMEM((nrow_idx, 128), jnp.float32))
    scratch += [
        pltpu.VMEM((KK * 128, Dc), jnp.float32),
        pltpu.VMEM((KK * 128, Dc), jnp.float32),
        pltpu.VMEM((KK * 128, Dc), jnp.float32),
        pltpu.VMEM((zrows, Dc), jnp.float32),
        pltpu.VMEM_SHARED((npad, Dc), jnp.float32),
    ] + [pltpu.SemaphoreType.DMA] * 6

    return pl.kernel(
        body,
        out_type=jax.ShapeDtypeStruct((NC, P, npad, Dc), jnp.float32),
        mesh=_mesh(),
        scratch_types=scratch,
        compiler_params=pltpu.CompilerParams(use_tc_tiling_on_sc=False),
    )


_SCATTER_CACHE = {}


def _scatter(xs2, srcp, dst, w, cfg):
    key = (cfg["n"], cfg["Dc"], cfg["P"], cfg["nch"], w is not None)
    if key not in _SCATTER_CACHE:
        _SCATTER_CACHE[key] = _make_scatter(
            cfg["n"], cfg["npad"], cfg["Dc"], cfg["P"], cfg["nch"],
            w is not None)
    f = _SCATTER_CACHE[key]
    if w is not None:
        return f(xs2, srcp, dst, w)
    return f(xs2, srcp, dst)


_RB = 1000


def _proj_kernel(x_ref, wt_ref, b_ref, y_ref):
    y_ref[...] = jnp.dot(x_ref[...], wt_ref[...],
                         preferred_element_type=jnp.float32) + b_ref[...]


def _tc_proj(x, W, b):
    n = x.shape[0]
    return pl.pallas_call(
        _proj_kernel,
        grid=(n // _RB,),
        in_specs=[
            pl.BlockSpec((_RB, D), lambda i: (i, 0)),
            pl.BlockSpec((D, D), lambda i: (0, 0)),
            pl.BlockSpec((1, D), lambda i: (0, 0)),
        ],
        out_specs=pl.BlockSpec((_RB, D), lambda i: (i, 0)),
        out_shape=jax.ShapeDtypeStruct((n, D), jnp.float32),
    )(x, W.T, b[None, :])


def _split_cols(xs, xs_ref, P, Dc):
    for p in range(P):
        xs_ref[:, p, :] = xs[:, p * Dc:(p + 1) * Dc]


def _pre_kernel(P, Dc, x_ref, dp1_ref, wt_ref, xw_ref, xs_ref):
    xw = jnp.dot(x_ref[...], wt_ref[...], preferred_element_type=jnp.float32)
    dis = lax.rsqrt(dp1_ref[...])
    xw_ref[...] = xw
    _split_cols(xw * dis, xs_ref, P, Dc)


def _tc_pre(x, dp1, W, cfg):
    n, P, Dc = cfg["n"], cfg["P"], cfg["Dc"]
    f = functools.partial(_pre_kernel, P, Dc)
    return pl.pallas_call(
        f,
        grid=(n // _RB,),
        in_specs=[
            pl.BlockSpec((_RB, D), lambda i: (i, 0)),
            pl.BlockSpec((_RB, 1), lambda i: (i, 0)),
            pl.BlockSpec((D, D), lambda i: (0, 0)),
        ],
        out_specs=[
            pl.BlockSpec((_RB, D), lambda i: (i, 0)),
            pl.BlockSpec((_RB, P, Dc), lambda i: (i, 0, 0)),
        ],
        out_shape=[
            jax.ShapeDtypeStruct((n, D), jnp.float32),
            jax.ShapeDtypeStruct((n, P, Dc), jnp.float32),
        ],
    )(x, dp1, W.T)


def _assemble(parts_ref, P, Dc):
    psum = parts_ref[0] + parts_ref[1]
    return jnp.concatenate([psum[p] for p in range(P)], axis=1)


def _post_kernel(P, Dc, parts_ref, xw_ref, dp1_ref, b_ref, h_ref):
    dis = lax.rsqrt(dp1_ref[...])
    agg = _assemble(parts_ref, P, Dc)
    h_ref[...] = jnp.maximum(
        dis * agg + dis * dis * xw_ref[...] + b_ref[...], 0.0)


def _tc_post(parts, xw, dp1, b, cfg):
    n, P, Dc = cfg["n"], cfg["P"], cfg["Dc"]
    f = functools.partial(_post_kernel, P, Dc)
    return pl.pallas_call(
        f,
        grid=(n // _RB,),
        in_specs=[
            pl.BlockSpec((2, P, _RB, Dc), lambda i: (0, 0, i, 0)),
            pl.BlockSpec((_RB, D), lambda i: (i, 0)),
            pl.BlockSpec((_RB, 1), lambda i: (i, 0)),
            pl.BlockSpec((1, D), lambda i: (0, 0)),
        ],
        out_specs=pl.BlockSpec((_RB, D), lambda i: (i, 0)),
        out_shape=jax.ShapeDtypeStruct((n, D), jnp.float32),
    )(parts, xw, dp1, b[None, :])


def _postpre_kernel(P, Dc, parts_ref, xw_ref, dp1_ref, b_ref, wt_ref,
                    xw2_ref, xs_ref):
    dis = lax.rsqrt(dp1_ref[...])
    agg = _assemble(parts_ref, P, Dc)
    x2 = jnp.maximum(dis * agg + dis * dis * xw_ref[...] + b_ref[...], 0.0)
    xw2 = jnp.dot(x2, wt_ref[...], preferred_element_type=jnp.float32)
    xw2_ref[...] = xw2
    _split_cols(xw2 * dis, xs_ref, P, Dc)


def _tc_postpre(parts, xw, dp1, b, Wn, cfg):
    n, P, Dc = cfg["n"], cfg["P"], cfg["Dc"]
    f = functools.partial(_postpre_kernel, P, Dc)
    return pl.pallas_call(
        f,
        grid=(n // _RB,),
        in_specs=[
            pl.BlockSpec((2, P, _RB, Dc), lambda i: (0, 0, i, 0)),
            pl.BlockSpec((_RB, D), lambda i: (i, 0)),
            pl.BlockSpec((_RB, 1), lambda i: (i, 0)),
            pl.BlockSpec((1, D), lambda i: (0, 0)),
            pl.BlockSpec((D, D), lambda i: (0, 0)),
        ],
        out_specs=[
            pl.BlockSpec((_RB, D), lambda i: (i, 0)),
            pl.BlockSpec((_RB, P, Dc), lambda i: (i, 0, 0)),
        ],
        out_shape=[
            jax.ShapeDtypeStruct((n, D), jnp.float32),
            jax.ShapeDtypeStruct((n, P, Dc), jnp.float32),
        ],
    )(parts, xw, dp1, b[None, :], Wn.T)


def _att_kernel(z0_ref, z1_ref, w1t_ref, b1_ref, w2t_ref, h_ref):
    a0 = jnp.dot(jnp.tanh(
        jnp.dot(z0_ref[...], w1t_ref[...],
                preferred_element_type=jnp.float32) + b1_ref[...]),
        w2t_ref[...], preferred_element_type=jnp.float32)
    a1 = jnp.dot(jnp.tanh(
        jnp.dot(z1_ref[...], w1t_ref[...],
                preferred_element_type=jnp.float32) + b1_ref[...]),
        w2t_ref[...], preferred_element_type=jnp.float32)
    m = jnp.maximum(a0, a1)
    e0 = jnp.exp(a0 - m)
    e1 = jnp.exp(a1 - m)
    h_ref[...] = (e0 * z0_ref[...] + e1 * z1_ref[...]) / (e0 + e1)


def _tc_attention(z0, z1, p):
    n = z0.shape[0]
    return pl.pallas_call(
        _att_kernel,
        grid=(n // _RB,),
        in_specs=[
            pl.BlockSpec((_RB, D), lambda i: (i, 0)),
            pl.BlockSpec((_RB, D), lambda i: (i, 0)),
            pl.BlockSpec((D, H_AT), lambda i: (0, 0)),
            pl.BlockSpec((1, H_AT), lambda i: (0, 0)),
            pl.BlockSpec((H_AT, 1), lambda i: (0, 0)),
        ],
        out_specs=pl.BlockSpec((_RB, D), lambda i: (i, 0)),
        out_shape=jax.ShapeDtypeStruct((n, D), jnp.float32),
    )(z0, z1, p["W1"].T, p["b1"][None, :], p["W2"].T)


def _prep_edges(ei, ew, cfg):
    n, P, nch = cfg["n"], cfg["P"], cfg["nch"]
    E = ei.shape[1]
    e_pad = NW * nch * 128 - E
    srcp = jnp.concatenate(
        [ei[0] * P, jnp.zeros((e_pad,), jnp.int32)]).reshape(NW, nch, 128)
    dstp = jnp.concatenate(
        [ei[1], jnp.full((e_pad,), n, jnp.int32)]).reshape(NW, nch, 128)
    zrow = jnp.zeros((NW, KK, 128), jnp.int32)
    srcp = jnp.concatenate([srcp, zrow], axis=1)
    dstp = jnp.concatenate([dstp, jnp.full((NW, KK, 128), n, jnp.int32)],
                           axis=1)
    if ew is None:
        return srcp, dstp, None
    wp = jnp.concatenate(
        [ew, jnp.zeros((e_pad,), jnp.float32)]).reshape(NW, nch, 128)
    wp = jnp.concatenate([wp, jnp.zeros((NW, KK, 128), jnp.float32)], axis=1)
    return srcp, dstp, wp


def _encoder(x0, layers, dp1, srcp, dstp, wp, cfg):
    n, P, Dc = cfg["n"], cfg["P"], cfg["Dc"]
    xw, xs = _tc_pre(x0, dp1, layers[0]["W"], cfg)
    for i in range(3):
        parts = _scatter(xs.reshape(n * P, Dc), srcp, dstp, wp, cfg)
        if i < 2:
            xw, xs = _tc_postpre(parts, xw, dp1, layers[i]["b"],
                                 layers[i + 1]["W"], cfg)
        else:
            return _tc_post(parts, xw, dp1, layers[i]["b"], cfg)


def _mlp_head_kernel(u_ref, v_ref, y_ref, w1u_ref, w1v_ref, b1_ref, g_ref,
                     be_ref, w2_ref, b2_ref, o_ref, l_ref):
    h = (jnp.dot(u_ref[...], w1u_ref[...], preferred_element_type=jnp.float32)
         + jnp.dot(v_ref[...], w1v_ref[...],
                   preferred_element_type=jnp.float32)) + b1_ref[...]
    h = h * g_ref[...] + be_ref[...]
    h = jnp.maximum(h, 0.0)
    o = jnp.dot(h, w2_ref[...], preferred_element_type=jnp.float32)
    o = jax.nn.sigmoid(o + b2_ref[0, 0])
    o_ref[...] = o
    oc = jnp.clip(o, 1e-7, 1.0 - 1e-7)
    y = y_ref[...]
    terms = y * jnp.log(oc) + (1.0 - y) * jnp.log(1.0 - oc)
    l_ref[...] = jnp.full((8, 128), jnp.sum(terms), jnp.float32)


def _mlp_head(u, v, labels, p):
    bn = p["gamma"] / jnp.sqrt(1.0 + 1e-5)
    w1 = p["W1"].T
    nb = B // 2048
    out, lsum = pl.pallas_call(
        _mlp_head_kernel,
        grid=(nb,),
        in_specs=[
            pl.BlockSpec((2048, D), lambda i: (i, 0)),
            pl.BlockSpec((2048, D), lambda i: (i, 0)),
            pl.BlockSpec((2048, 1), lambda i: (i, 0)),
            pl.BlockSpec((D, D), lambda i: (0, 0)),
            pl.BlockSpec((D, D), lambda i: (0, 0)),
            pl.BlockSpec((1, D), lambda i: (0, 0)),
            pl.BlockSpec((1, D), lambda i: (0, 0)),
            pl.BlockSpec((1, D), lambda i: (0, 0)),
            pl.BlockSpec((D, 1), lambda i: (0, 0)),
            pl.BlockSpec((1, 1), lambda i: (0, 0)),
        ],
        out_specs=[
            pl.BlockSpec((2048, 1), lambda i: (i, 0)),
            pl.BlockSpec((8, 128), lambda i: (i, 0)),
        ],
        out_shape=[
            jax.ShapeDtypeStruct((B, 1), jnp.float32),
            jax.ShapeDtypeStruct((nb * 8, 128), jnp.float32),
        ],
    )(u, v, labels[:, None], w1[:D], w1[D:], p["b1"][None, :], bn[None, :],
      p["beta"][None, :], p["W2"].T, p["b2"][None, :])
    return out[:, 0], jnp.sum(lsum[::8, 0])


def kernel(xr, xp, sr, sp, params, rp_edge_index, rr_edge_index,
           pp_edge_index, samples, labels):
    rr_s, rr_d, rr_w = _prep_edges(rr_edge_index, sr, _RR_CFG)
    pp_s, pp_d, pp_w = _prep_edges(pp_edge_index, sp, _RR_CFG)
    rp_s, rp_d, _ = _prep_edges(rp_edge_index, None, _RP_CFG)

    def _deg_slice(ei, ew, cfg, off):
        E = ei.shape[1]
        e_pad = NW * cfg["nch"] * 128 - E
        dd = jnp.concatenate(
            [ei[1], jnp.full((e_pad,), cfg["n"], jnp.int32)]) + off
        if ew is None:
            ww = jnp.concatenate(
                [jnp.ones((E,), jnp.float32), jnp.zeros((e_pad,))])
        else:
            ww = jnp.concatenate([ew, jnp.zeros((e_pad,), jnp.float32)])
        return dd, ww

    d0, w0 = _deg_slice(rr_edge_index, sr, _RR_CFG, _DEG_OFF[0])
    d1, w1 = _deg_slice(pp_edge_index, sp, _RR_CFG, _DEG_OFF[1])
    d2, w2 = _deg_slice(rp_edge_index, None, _RP_CFG, _DEG_OFF[2])
    dst_cat = jnp.concatenate([d0, d1, d2]).reshape(NW, _DEG_NCH, 128)
    w_cat = jnp.concatenate([w0, w1, w2]).reshape(NW, _DEG_NCH, 128)
    degp = _sc_degrees(dst_cat, w_cat)
    deg = degp[0] + degp[1]
    dp1_rr = (deg[_DEG_OFF[0]:_DEG_OFF[0] + NR] + 1.0)[:, None]
    dp1_pp = (deg[_DEG_OFF[1]:_DEG_OFF[1] + NP_] + 1.0)[:, None]
    dp1_rp = (deg[_DEG_OFF[2]:_DEG_OFF[2] + N] + 1.0)[:, None]

    rr_xr = _tc_proj(xr, params["rr_xr_proj"]["W"], params["rr_xr_proj"]["b"])
    pp_xp = _tc_proj(xp, params["pp_xp_proj"]["W"], params["pp_xp_proj"]["b"])
    rp_xr = _tc_proj(xr, params["rp_xr_proj"]["W"], params["rp_xr_proj"]["b"])
    rp_xp = _tc_proj(xp, params["rp_xp_proj"]["W"], params["rp_xp_proj"]["b"])
    rp_x = jnp.concatenate([rp_xr, rp_xp])

    rr_hr = _encoder(rr_xr, params["rr_enc"], dp1_rr, rr_s, rr_d, rr_w,
                     _RR_CFG)
    rp_h = _encoder(rp_x, params["rp_enc"], dp1_rp, rp_s, rp_d, None,
                    _RP_CFG)
    pp_hp = _encoder(pp_xp, params["pp_enc"], dp1_pp, pp_s, pp_d, pp_w,
                     _RR_CFG)
    rp_hr = rp_h[:NR]
    rp_hp = rp_h[NR:]

    hr = _tc_attention(rr_hr, rp_hr, params["r_at"])
    hp = _tc_attention(pp_hp, rp_hp, params["p_at"])
    z = jnp.concatenate([hr, hp])
    s0 = samples[:, 0]
    s1 = samples[:, 1]
    out, ls = _mlp_head(z[s0], z[s1], labels, params["mlp"])
    out_i, ls_i = _mlp_head(rp_h[s0], rp_h[s1], labels, params["i_mlp"])
    z_c = jnp.concatenate([rr_hr, pp_hp])
    out_c, ls_c = _mlp_head(z_c[s0], z_c[s1], labels, params["c_mlp"])
    loss = -(ls + 0.5 * (ls_i + ls_c)) / B
    return (out, loss, rr_hr, rp_hr, pp_hp, rp_hp)
